# Initial kernel scaffold; baseline (speedup 1.0000x reference)
import jax
import jax.numpy as jnp
from jax.experimental import pallas as pl


def kernel(feat_0, feat_1, feat_2, feat_3, feat_4, feat_5, feat_6, feat_7, feat_8, W):
    raise NotImplementedError("write your pallas kernel here")


# speedup vs baseline: 4.1736x; 4.1736x over previous
import jax
import jax.numpy as jnp
from jax.experimental import pallas as pl


def _fuse_body(f0, f1, f2, f3, f4, f5, f6, f7, f8, w_ref, out_ref):
    w = w_ref[0, :][None, :]
    feats = [r[...] for r in (f0, f1, f2, f3, f4, f5, f6, f7, f8)]
    scores = [jnp.sum(f * w, axis=1, keepdims=True) for f in feats]
    m = scores[0]
    for s in scores[1:]:
        m = jnp.maximum(m, s)
    exps = [jnp.exp(s - m) for s in scores]
    denom = exps[0]
    for e in exps[1:]:
        denom = denom + e
    inv = 1.0 / denom
    acc = feats[0] * (exps[0] * inv)
    for i in range(1, 9):
        acc = acc + feats[i] * (exps[i] * inv)
    out_ref[...] = acc


def kernel(feat_0, feat_1, feat_2, feat_3, feat_4, feat_5, feat_6, feat_7,
           feat_8, W):
    S, B, H = feat_0.shape
    R = S * B
    blk = min(1024, R)
    feats = [f.reshape(R, H) for f in (feat_0, feat_1, feat_2, feat_3, feat_4,
                                       feat_5, feat_6, feat_7, feat_8)]
    w2 = W.reshape(1, H)
    feat_spec = pl.BlockSpec((blk, H), lambda i: (i, 0))
    out = pl.pallas_call(
        _fuse_body,
        grid=(R // blk,),
        in_specs=[feat_spec] * 9 + [pl.BlockSpec((1, H), lambda i: (0, 0))],
        out_specs=feat_spec,
        out_shape=jax.ShapeDtypeStruct((R, H), feat_0.dtype),
    )(*feats, w2)
    return out.reshape(S, B, H)


# speedup vs baseline: 4.2100x; 1.0087x over previous
import jax
import jax.numpy as jnp
from jax.experimental import pallas as pl


def _fuse_body(f0, f1, f2, f3, f4, f5, f6, f7, f8, w_ref, out_ref):
    w = w_ref[0, :][None, :]
    feats = [r[...] for r in (f0, f1, f2, f3, f4, f5, f6, f7, f8)]
    scores = [jnp.sum(f * w, axis=1, keepdims=True) for f in feats]
    m = scores[0]
    for s in scores[1:]:
        m = jnp.maximum(m, s)
    exps = [jnp.exp(s - m) for s in scores]
    denom = exps[0]
    for e in exps[1:]:
        denom = denom + e
    inv = 1.0 / denom
    acc = feats[0] * (exps[0] * inv)
    for i in range(1, 9):
        acc = acc + feats[i] * (exps[i] * inv)
    out_ref[...] = acc


def kernel(feat_0, feat_1, feat_2, feat_3, feat_4, feat_5, feat_6, feat_7,
           feat_8, W):
    S, B, H = feat_0.shape
    R = S * B
    blk = min(512, R)
    feats = [f.reshape(R, H) for f in (feat_0, feat_1, feat_2, feat_3, feat_4,
                                       feat_5, feat_6, feat_7, feat_8)]
    w2 = W.reshape(1, H)
    feat_spec = pl.BlockSpec((blk, H), lambda i: (i, 0))
    out = pl.pallas_call(
        _fuse_body,
        grid=(R // blk,),
        in_specs=[feat_spec] * 9 + [pl.BlockSpec((1, H), lambda i: (0, 0))],
        out_specs=feat_spec,
        out_shape=jax.ShapeDtypeStruct((R, H), feat_0.dtype),
    )(*feats, w2)
    return out.reshape(S, B, H)
